# Initial kernel scaffold; baseline (speedup 1.0000x reference)
import jax
import jax.numpy as jnp
from jax.experimental import pallas as pl


def kernel(ids, feats, edge_dict, epo, vcn_kk_w, vcn_kk_b, vcn_k1_w, vcn_k1_b, ec_w1, ec_b1, ec_w2, ec_b2, fc_w, fc_b):
    raise NotImplementedError("write your pallas kernel here")


# speedup vs baseline: 3.2369x; 3.2369x over previous
import jax
import jax.numpy as jnp
from jax.experimental import pallas as pl
from jax.experimental.pallas import tpu as pltpu

N = 10000
D = 128
KN = 32
RB = 64
NRP = ((N + RB - 1) // RB) * RB
NP = 10240
RB2 = 256
NVP = ((N + RB2 - 1) // RB2) * RB2


_NEG = -3.0e38


def _simtopk_body(fn_ref, fnt_ref, idx_ref, s_ref):
    s = jax.lax.dot_general(fn_ref[...], fnt_ref[...],
                            (((1,), (0,)), ((), ())),
                            preferred_element_type=jnp.float32)
    col = jax.lax.broadcasted_iota(jnp.int32, s.shape, 1)
    s_ref[...] = jnp.where(col < N, s, _NEG)

    kcol = jax.lax.broadcasted_iota(jnp.int32, (RB, KN), 1)

    def body(k, acc):
        s = s_ref[...]
        m = jnp.max(s, axis=1, keepdims=True)
        cand = jnp.where(s == m, col, NP)
        c = jnp.min(cand, axis=1, keepdims=True)
        acc = jnp.where(kcol == k, c, acc)
        s_ref[...] = jnp.where(col == c, _NEG, s)
        return acc

    idx_ref[...] = jax.lax.fori_loop(
        0, KN, body, jnp.zeros((RB, KN), jnp.int32))


def _simtopk(fn, fnt):
    return pl.pallas_call(
        _simtopk_body,
        grid=(NRP // RB,),
        in_specs=[
            pl.BlockSpec((RB, D), lambda i: (i, 0)),
            pl.BlockSpec((D, NP), lambda i: (0, 0)),
        ],
        out_specs=pl.BlockSpec((RB, KN), lambda i: (i, 0)),
        out_shape=jax.ShapeDtypeStruct((NRP, KN), jnp.int32),
        scratch_shapes=[pltpu.VMEM((RB, NP), jnp.float32)],
    )(fn, fnt)


def kernel(ids, feats, edge_dict, epo, vcn_kk_w, vcn_kk_b, vcn_k1_w, vcn_k1_b,
           ec_w1, ec_b1, ec_w2, ec_b2, fc_w, fc_b):
    norms = jnp.sqrt(jnp.sum(feats * feats, axis=1, keepdims=True)) + 1e-12
    fn = feats / norms
    fn_pad = jnp.pad(fn, ((0, NRP - N), (0, 0)))
    fnt = jnp.pad(fn.T, ((0, 0), (0, NP - N)))
    idx_full = _simtopk(fn_pad, fnt)
    idx = idx_full[:N][ids]

    nearest = feats[idx.reshape(-1)].reshape(N, KN, D)
    Wr = vcn_kk_w.reshape(KN, KN, D)
    convd = jnp.einsum('nid,ijd->nij', nearest, Wr) + vcn_kk_b.reshape(KN, KN)
    mult = jax.nn.softmax(convd, axis=-1)
    cw = jnp.einsum('i,nij->nj', vcn_k1_w, mult)
    pooled = jnp.einsum('nj,njd->nd', cw, nearest) + vcn_k1_b
    out = jnp.maximum(pooled @ fc_w.T + fc_b, 0.0)
    return out


# speedup vs baseline: 6.5650x; 2.0281x over previous
import jax
import jax.numpy as jnp
from jax.experimental import pallas as pl
from jax.experimental.pallas import tpu as pltpu

N = 10000
D = 128
KN = 32
RB = 128
NRP = ((N + RB - 1) // RB) * RB
NP = 10240
G = 256
P = NP // G
RB2 = 256
NVP = ((N + RB2 - 1) // RB2) * RB2


_NEG = -3.0e38
_BIG = NP


def _insert3(state, v, cv):
    g1, c1, g2, c2, g3, c3 = state
    w1 = v > g1
    lv = jnp.where(w1, g1, v)
    lc = jnp.where(w1, c1, cv)
    g1 = jnp.where(w1, v, g1)
    c1 = jnp.where(w1, cv, c1)
    w2 = lv > g2
    lv2 = jnp.where(w2, g2, lv)
    lc2 = jnp.where(w2, c2, lc)
    g2 = jnp.where(w2, lv, g2)
    c2 = jnp.where(w2, lc, c2)
    w3 = lv2 > g3
    g3 = jnp.where(w3, lv2, g3)
    c3 = jnp.where(w3, lc2, c3)
    return (g1, c1, g2, c2, g3, c3)


def _init3():
    return (jnp.full((RB, G), _NEG, jnp.float32),
            jnp.full((RB, G), _BIG, jnp.int32),
            jnp.full((RB, G), _NEG, jnp.float32),
            jnp.full((RB, G), _BIG, jnp.int32),
            jnp.full((RB, G), _NEG, jnp.float32),
            jnp.full((RB, G), _BIG, jnp.int32))


def _simtopk_body(fn_ref, fnt_ref, idx_ref, s_ref,
                  g1_ref, c1_ref, g2_ref, c2_ref, g3_ref, c3_ref, cnt_ref):
    fnb = fn_ref[...]
    gidx = jax.lax.broadcasted_iota(jnp.int32, (RB, G), 1)

    st = _init3()
    for p in range(P):
        v = jax.lax.dot_general(fnb, fnt_ref[:, p * G:(p + 1) * G],
                                (((1,), (0,)), ((), ())),
                                preferred_element_type=jnp.float32)
        cv = gidx + (p * G)
        v = jnp.where(cv < N, v, _NEG)
        s_ref[:, p * G:(p + 1) * G] = v
        st = _insert3(st, v, cv)
    g1_ref[...], c1_ref[...], g2_ref[...], c2_ref[...], \
        g3_ref[...], c3_ref[...] = st
    cnt_ref[...] = jnp.full((RB, G), 3, jnp.int32)

    kcol = jax.lax.broadcasted_iota(jnp.int32, (RB, KN), 1)

    def ext(k, acc):
        g1 = g1_ref[...]
        c1 = c1_ref[...]
        m = jnp.max(g1, axis=1, keepdims=True)
        c = jnp.min(jnp.where(g1 == m, c1, _BIG), axis=1, keepdims=True)
        acc = jnp.where(kcol == k, c, acc)
        hit = (g1 == m) & (c1 == c)
        cnt = cnt_ref[...]
        need = jnp.any(hit & (cnt <= 1))

        @pl.when(need)
        def _rebuild():
            st = _init3()
            for p in range(P):
                v = s_ref[:, p * G:(p + 1) * G]
                cv = gidx + (p * G)
                alive = (v < m) | ((v == m) & (cv > c))
                v = jnp.where(alive, v, _NEG)
                st = _insert3(st, v, cv)
            g1_ref[...], c1_ref[...], g2_ref[...], c2_ref[...], \
                g3_ref[...], c3_ref[...] = st
            cnt_ref[...] = jnp.full((RB, G), 3, jnp.int32)

        @pl.when(jnp.logical_not(need))
        def _promote():
            g1_ref[...] = jnp.where(hit, g2_ref[...], g1)
            c1_ref[...] = jnp.where(hit, c2_ref[...], c1)
            g2_ref[...] = jnp.where(hit, g3_ref[...], g2_ref[...])
            c2_ref[...] = jnp.where(hit, c3_ref[...], c2_ref[...])
            cnt_ref[...] = jnp.where(hit, cnt - 1, cnt)

        return acc

    idx_ref[...] = jax.lax.fori_loop(
        0, KN, ext, jnp.zeros((RB, KN), jnp.int32))


def _simtopk(fn, fnt):
    return pl.pallas_call(
        _simtopk_body,
        grid=(NRP // RB,),
        in_specs=[
            pl.BlockSpec((RB, D), lambda i: (i, 0)),
            pl.BlockSpec((D, NP), lambda i: (0, 0)),
        ],
        out_specs=pl.BlockSpec((RB, KN), lambda i: (i, 0)),
        out_shape=jax.ShapeDtypeStruct((NRP, KN), jnp.int32),
        scratch_shapes=[
            pltpu.VMEM((RB, NP), jnp.float32),
            pltpu.VMEM((RB, G), jnp.float32),
            pltpu.VMEM((RB, G), jnp.int32),
            pltpu.VMEM((RB, G), jnp.float32),
            pltpu.VMEM((RB, G), jnp.int32),
            pltpu.VMEM((RB, G), jnp.float32),
            pltpu.VMEM((RB, G), jnp.int32),
            pltpu.VMEM((RB, G), jnp.int32),
        ],
    )(fn, fnt)


def kernel(ids, feats, edge_dict, epo, vcn_kk_w, vcn_kk_b, vcn_k1_w, vcn_k1_b,
           ec_w1, ec_b1, ec_w2, ec_b2, fc_w, fc_b):
    norms = jnp.sqrt(jnp.sum(feats * feats, axis=1, keepdims=True)) + 1e-12
    fn = feats / norms
    fn_pad = jnp.pad(fn, ((0, NRP - N), (0, 0)))
    fnt = jnp.pad(fn.T, ((0, 0), (0, NP - N)))
    idx_full = _simtopk(fn_pad, fnt)
    idx = idx_full[:N][ids]

    nearest = feats[idx.reshape(-1)].reshape(N, KN, D)
    Wr = vcn_kk_w.reshape(KN, KN, D)
    convd = jnp.einsum('nid,ijd->nij', nearest, Wr) + vcn_kk_b.reshape(KN, KN)
    mult = jax.nn.softmax(convd, axis=-1)
    cw = jnp.einsum('i,nij->nj', vcn_k1_w, mult)
    pooled = jnp.einsum('nj,njd->nd', cw, nearest) + vcn_k1_b
    out = jnp.maximum(pooled @ fc_w.T + fc_b, 0.0)
    return out


# speedup vs baseline: 6.9903x; 1.0648x over previous
import functools

import jax
import jax.numpy as jnp
from jax import lax
from jax.experimental import pallas as pl
from jax.experimental.pallas import tpu as pltpu
from jax.experimental.pallas import tpu_sc as plsc

N = 10000
D = 128
KN = 32
RB = 128
NRP = ((N + RB - 1) // RB) * RB
NP = 10240
G = 256
P = NP // G
RB2 = 256
NVP = ((N + RB2 - 1) // RB2) * RB2


_NEG = -3.0e38
_BIG = NP


def _insert3(state, v, cv):
    g1, c1, g2, c2, g3, c3 = state
    w1 = v > g1
    lv = jnp.where(w1, g1, v)
    lc = jnp.where(w1, c1, cv)
    g1 = jnp.where(w1, v, g1)
    c1 = jnp.where(w1, cv, c1)
    w2 = lv > g2
    lv2 = jnp.where(w2, g2, lv)
    lc2 = jnp.where(w2, c2, lc)
    g2 = jnp.where(w2, lv, g2)
    c2 = jnp.where(w2, lc, c2)
    w3 = lv2 > g3
    g3 = jnp.where(w3, lv2, g3)
    c3 = jnp.where(w3, lc2, c3)
    return (g1, c1, g2, c2, g3, c3)


def _init3():
    return (jnp.full((RB, G), _NEG, jnp.float32),
            jnp.full((RB, G), _BIG, jnp.int32),
            jnp.full((RB, G), _NEG, jnp.float32),
            jnp.full((RB, G), _BIG, jnp.int32),
            jnp.full((RB, G), _NEG, jnp.float32),
            jnp.full((RB, G), _BIG, jnp.int32))


def _simtopk_body(fn_ref, fnt_ref, idx_ref, s_ref,
                  g1_ref, c1_ref, g2_ref, c2_ref, g3_ref, c3_ref, cnt_ref):
    fnb = fn_ref[...]
    gidx = jax.lax.broadcasted_iota(jnp.int32, (RB, G), 1)

    st = _init3()
    for p in range(P):
        v = jax.lax.dot_general(fnb, fnt_ref[:, p * G:(p + 1) * G],
                                (((1,), (0,)), ((), ())),
                                preferred_element_type=jnp.float32)
        cv = gidx + (p * G)
        v = jnp.where(cv < N, v, _NEG)
        s_ref[:, p * G:(p + 1) * G] = v
        st = _insert3(st, v, cv)
    g1_ref[...], c1_ref[...], g2_ref[...], c2_ref[...], \
        g3_ref[...], c3_ref[...] = st
    cnt_ref[...] = jnp.full((RB, G), 3, jnp.int32)

    kcol = jax.lax.broadcasted_iota(jnp.int32, (RB, KN), 1)

    def ext(k, acc):
        g1 = g1_ref[...]
        c1 = c1_ref[...]
        m = jnp.max(g1, axis=1, keepdims=True)
        c = jnp.min(jnp.where(g1 == m, c1, _BIG), axis=1, keepdims=True)
        acc = jnp.where(kcol == k, c, acc)
        hit = (g1 == m) & (c1 == c)
        cnt = cnt_ref[...]
        need = jnp.any(hit & (cnt <= 1))

        @pl.when(need)
        def _rebuild():
            st = _init3()
            for p in range(P):
                v = s_ref[:, p * G:(p + 1) * G]
                cv = gidx + (p * G)
                alive = (v < m) | ((v == m) & (cv > c))
                v = jnp.where(alive, v, _NEG)
                st = _insert3(st, v, cv)
            g1_ref[...], c1_ref[...], g2_ref[...], c2_ref[...], \
                g3_ref[...], c3_ref[...] = st
            cnt_ref[...] = jnp.full((RB, G), 3, jnp.int32)

        @pl.when(jnp.logical_not(need))
        def _promote():
            g1_ref[...] = jnp.where(hit, g2_ref[...], g1)
            c1_ref[...] = jnp.where(hit, c2_ref[...], c1)
            g2_ref[...] = jnp.where(hit, g3_ref[...], g2_ref[...])
            c2_ref[...] = jnp.where(hit, c3_ref[...], c2_ref[...])
            cnt_ref[...] = jnp.where(hit, cnt - 1, cnt)

        return acc

    idx_ref[...] = jax.lax.fori_loop(
        0, KN, ext, jnp.zeros((RB, KN), jnp.int32))


def _simtopk(fn, fnt):
    return pl.pallas_call(
        _simtopk_body,
        grid=(NRP // RB,),
        in_specs=[
            pl.BlockSpec((RB, D), lambda i: (i, 0)),
            pl.BlockSpec((D, NP), lambda i: (0, 0)),
        ],
        out_specs=pl.BlockSpec((RB, KN), lambda i: (i, 0)),
        out_shape=jax.ShapeDtypeStruct((NRP, KN), jnp.int32),
        scratch_shapes=[
            pltpu.VMEM((RB, NP), jnp.float32),
            pltpu.VMEM((RB, G), jnp.float32),
            pltpu.VMEM((RB, G), jnp.int32),
            pltpu.VMEM((RB, G), jnp.float32),
            pltpu.VMEM((RB, G), jnp.int32),
            pltpu.VMEM((RB, G), jnp.float32),
            pltpu.VMEM((RB, G), jnp.int32),
            pltpu.VMEM((RB, G), jnp.int32),
        ],
    )(fn, fnt)


def _vconv_body(g_ref, wr_ref, kkb_ref, w1_ref, k1b_ref, fcwt_ref, fcb_ref,
                out_ref):
    kkb = kkb_ref[...]
    cw = jnp.zeros((RB2, KN), jnp.float32)
    for i in range(KN):
        convd = jax.lax.dot_general(
            g_ref[i], wr_ref[i], (((1,), (1,)), ((), ())),
            preferred_element_type=jnp.float32)
        convd = convd + kkb[i:i + 1, :]
        mx = jnp.max(convd, axis=-1, keepdims=True)
        e = jnp.exp(convd - mx)
        mult = e / jnp.sum(e, axis=-1, keepdims=True)
        cw = cw + mult * w1_ref[i]
    pooled = jnp.zeros((RB2, D), jnp.float32)
    for j in range(KN):
        pooled = pooled + cw[:, j:j + 1] * g_ref[j]
    pooled = pooled + k1b_ref[0]
    out = jnp.dot(pooled, fcwt_ref[...],
                  preferred_element_type=jnp.float32) + fcb_ref[...]
    out_ref[...] = jnp.maximum(out, 0.0)


def _vconv(g, wr, kkb, w1, k1b, fcwt, fcb):
    return pl.pallas_call(
        _vconv_body,
        grid=(NVP // RB2,),
        in_specs=[
            pl.BlockSpec((KN, RB2, D), lambda i: (0, i, 0)),
            pl.BlockSpec((KN, KN, D), lambda i: (0, 0, 0)),
            pl.BlockSpec((KN, KN), lambda i: (0, 0)),
            pl.BlockSpec(memory_space=pltpu.SMEM),
            pl.BlockSpec(memory_space=pltpu.SMEM),
            pl.BlockSpec((D, D), lambda i: (0, 0)),
            pl.BlockSpec((1, D), lambda i: (0, 0)),
        ],
        out_specs=pl.BlockSpec((RB2, D), lambda i: (i, 0)),
        out_shape=jax.ShapeDtypeStruct((NVP, D), jnp.float32),
    )(g, wr, kkb, w1, k1b, fcwt, fcb)


_NW = 32
_BG = KN * NVP
_BPW = _BG // _NW
_CG = 256


@functools.partial(
    pl.kernel,
    mesh=plsc.VectorSubcoreMesh(core_axis_name="c", subcore_axis_name="s"),
    out_type=jax.ShapeDtypeStruct((_BG, D), jnp.float32),
    scratch_types=[
        pltpu.VMEM((_BPW,), jnp.int32),
        pltpu.VMEM((_CG, D), jnp.float32),
        pltpu.SemaphoreType.DMA,
    ],
)
def _gather_sc(table_hbm, idx_hbm, out_hbm, idx_v, buf_v, sem):
    wid = lax.axis_index("s") * 2 + lax.axis_index("c")
    base = wid * _BPW
    pltpu.sync_copy(idx_hbm.at[pl.ds(base, _BPW)], idx_v)

    def chunk(ci, _):
        off = ci * _CG
        pltpu.async_copy(table_hbm.at[idx_v.at[pl.ds(off, _CG)]], buf_v,
                         sem).wait()
        pltpu.sync_copy(buf_v, out_hbm.at[pl.ds(base + off, _CG)])
        return 0

    jax.lax.fori_loop(0, _BPW // _CG, chunk, 0)


def kernel(ids, feats, edge_dict, epo, vcn_kk_w, vcn_kk_b, vcn_k1_w, vcn_k1_b,
           ec_w1, ec_b1, ec_w2, ec_b2, fc_w, fc_b):
    norms = jnp.sqrt(jnp.sum(feats * feats, axis=1, keepdims=True)) + 1e-12
    fn = feats / norms
    fn_pad = jnp.pad(fn, ((0, NRP - N), (0, 0)))
    fnt = jnp.pad(fn.T, ((0, 0), (0, NP - N)))
    idx_full = _simtopk(fn_pad, fnt)
    idx = idx_full[:N][ids]

    idx_pad = jnp.pad(idx, ((0, NVP - N), (0, 0)))
    g = _gather_sc(feats, idx_pad.T.reshape(-1)).reshape(KN, NVP, D)

    wr = vcn_kk_w.reshape(KN, KN, D)
    out_full = _vconv(g, wr, vcn_kk_b.reshape(KN, KN), vcn_k1_w, vcn_k1_b,
                      fc_w.T, fc_b.reshape(1, D))
    return out_full[:N]


# speedup vs baseline: 8.6871x; 1.2427x over previous
import functools

import jax
import jax.numpy as jnp
from jax import lax
from jax.experimental import pallas as pl
from jax.experimental.pallas import tpu as pltpu
from jax.experimental.pallas import tpu_sc as plsc

N = 10000
D = 128
KN = 32
RB = 128
NRP = ((N + RB - 1) // RB) * RB
NP = 10240
G = 256
P = NP // G
RB2 = 256
NVP = ((N + RB2 - 1) // RB2) * RB2


_NEG = -3.0e38
_BIG = NP


def _insert3(state, v, cv):
    g1, c1, g2, c2, g3, c3 = state
    w1 = v > g1
    lv = jnp.where(w1, g1, v)
    lc = jnp.where(w1, c1, cv)
    g1 = jnp.where(w1, v, g1)
    c1 = jnp.where(w1, cv, c1)
    w2 = lv > g2
    lv2 = jnp.where(w2, g2, lv)
    lc2 = jnp.where(w2, c2, lc)
    g2 = jnp.where(w2, lv, g2)
    c2 = jnp.where(w2, lc, c2)
    w3 = lv2 > g3
    g3 = jnp.where(w3, lv2, g3)
    c3 = jnp.where(w3, lc2, c3)
    return (g1, c1, g2, c2, g3, c3)


def _init3():
    return (jnp.full((G, RB), _NEG, jnp.float32),
            jnp.full((G, RB), _BIG, jnp.int32),
            jnp.full((G, RB), _NEG, jnp.float32),
            jnp.full((G, RB), _BIG, jnp.int32),
            jnp.full((G, RB), _NEG, jnp.float32),
            jnp.full((G, RB), _BIG, jnp.int32))


def _simtopk_body(fn_ref, fnb_ref, idxT_ref, s_ref,
                  g1_ref, c1_ref, g2_ref, c2_ref, g3_ref, c3_ref, cnt_ref):
    fnb = fnb_ref[...]
    gidx = jax.lax.broadcasted_iota(jnp.int32, (G, RB), 0)

    st = _init3()
    for p in range(P):
        v = jax.lax.dot_general(fn_ref[p * G:(p + 1) * G, :], fnb,
                                (((1,), (1,)), ((), ())),
                                preferred_element_type=jnp.float32)
        cv = gidx + (p * G)
        v = jnp.where(cv < N, v, _NEG)
        s_ref[p * G:(p + 1) * G, :] = v
        st = _insert3(st, v, cv)
    g1_ref[...], c1_ref[...], g2_ref[...], c2_ref[...], \
        g3_ref[...], c3_ref[...] = st
    cnt_ref[...] = jnp.full((G, RB), 3, jnp.int32)

    krow = jax.lax.broadcasted_iota(jnp.int32, (KN, RB), 0)

    def ext(k, accT):
        g1 = g1_ref[...]
        c1 = c1_ref[...]
        m = jnp.max(g1, axis=0, keepdims=True)
        c = jnp.min(jnp.where(g1 == m, c1, _BIG), axis=0, keepdims=True)
        accT = jnp.where(krow == k, c, accT)
        hit = (g1 == m) & (c1 == c)
        cnt = cnt_ref[...]
        need = jnp.any(hit & (cnt <= 1))

        @pl.when(need)
        def _rebuild():
            st = _init3()
            for p in range(P):
                v = s_ref[p * G:(p + 1) * G, :]
                cv = gidx + (p * G)
                alive = (v < m) | ((v == m) & (cv > c))
                v = jnp.where(alive, v, _NEG)
                st = _insert3(st, v, cv)
            g1_ref[...], c1_ref[...], g2_ref[...], c2_ref[...], \
                g3_ref[...], c3_ref[...] = st
            cnt_ref[...] = jnp.full((G, RB), 3, jnp.int32)

        @pl.when(jnp.logical_not(need))
        def _promote():
            g1_ref[...] = jnp.where(hit, g2_ref[...], g1)
            c1_ref[...] = jnp.where(hit, c2_ref[...], c1)
            g2_ref[...] = jnp.where(hit, g3_ref[...], g2_ref[...])
            c2_ref[...] = jnp.where(hit, c3_ref[...], c2_ref[...])
            cnt_ref[...] = jnp.where(hit, cnt - 1, cnt)

        return accT

    idxT_ref[...] = jax.lax.fori_loop(
        0, KN, ext, jnp.zeros((KN, RB), jnp.int32))


def _simtopk(fn):
    return pl.pallas_call(
        _simtopk_body,
        grid=(NRP // RB,),
        in_specs=[
            pl.BlockSpec((NP, D), lambda i: (0, 0)),
            pl.BlockSpec((RB, D), lambda i: (i, 0)),
        ],
        out_specs=pl.BlockSpec((KN, RB), lambda i: (0, i)),
        out_shape=jax.ShapeDtypeStruct((KN, NRP), jnp.int32),
        scratch_shapes=[
            pltpu.VMEM((NP, RB), jnp.float32),
            pltpu.VMEM((G, RB), jnp.float32),
            pltpu.VMEM((G, RB), jnp.int32),
            pltpu.VMEM((G, RB), jnp.float32),
            pltpu.VMEM((G, RB), jnp.int32),
            pltpu.VMEM((G, RB), jnp.float32),
            pltpu.VMEM((G, RB), jnp.int32),
            pltpu.VMEM((G, RB), jnp.int32),
        ],
    )(fn, fn)


def _vconv_body(g_ref, wr_ref, kkb_ref, w1_ref, k1b_ref, fcwt_ref, fcb_ref,
                out_ref):
    kkb = kkb_ref[...]
    cw = jnp.zeros((RB2, KN), jnp.float32)
    for i in range(KN):
        convd = jax.lax.dot_general(
            g_ref[i], wr_ref[i], (((1,), (1,)), ((), ())),
            preferred_element_type=jnp.float32)
        convd = convd + kkb[i:i + 1, :]
        mx = jnp.max(convd, axis=-1, keepdims=True)
        e = jnp.exp(convd - mx)
        mult = e / jnp.sum(e, axis=-1, keepdims=True)
        cw = cw + mult * w1_ref[i]
    pooled = jnp.zeros((RB2, D), jnp.float32)
    for j in range(KN):
        pooled = pooled + cw[:, j:j + 1] * g_ref[j]
    pooled = pooled + k1b_ref[0]
    out = jnp.dot(pooled, fcwt_ref[...],
                  preferred_element_type=jnp.float32) + fcb_ref[...]
    out_ref[...] = jnp.maximum(out, 0.0)


def _vconv(g, wr, kkb, w1, k1b, fcwt, fcb):
    return pl.pallas_call(
        _vconv_body,
        grid=(NVP // RB2,),
        in_specs=[
            pl.BlockSpec((KN, RB2, D), lambda i: (0, i, 0)),
            pl.BlockSpec((KN, KN, D), lambda i: (0, 0, 0)),
            pl.BlockSpec((KN, KN), lambda i: (0, 0)),
            pl.BlockSpec(memory_space=pltpu.SMEM),
            pl.BlockSpec(memory_space=pltpu.SMEM),
            pl.BlockSpec((D, D), lambda i: (0, 0)),
            pl.BlockSpec((1, D), lambda i: (0, 0)),
        ],
        out_specs=pl.BlockSpec((RB2, D), lambda i: (i, 0)),
        out_shape=jax.ShapeDtypeStruct((NVP, D), jnp.float32),
    )(g, wr, kkb, w1, k1b, fcwt, fcb)


_NW = 32
_BG = KN * NVP
_BPW = _BG // _NW
_CG = 256


@functools.cache
def _gather_sc_fn():
    @functools.partial(
        pl.kernel,
        mesh=plsc.VectorSubcoreMesh(core_axis_name="c", subcore_axis_name="s"),
        out_type=jax.ShapeDtypeStruct((_BG, D), jnp.float32),
        scratch_types=[
            pltpu.VMEM((_BPW,), jnp.int32),
            pltpu.VMEM((_CG, D), jnp.float32),
            pltpu.SemaphoreType.DMA,
        ],
    )
    def _gather_sc(table_hbm, idx_hbm, out_hbm, idx_v, buf_v, sem):
        wid = lax.axis_index("s") * 2 + lax.axis_index("c")
        base = wid * _BPW
        pltpu.sync_copy(idx_hbm.at[pl.ds(base, _BPW)], idx_v)

        def chunk(ci, _):
            off = ci * _CG
            pltpu.async_copy(table_hbm.at[idx_v.at[pl.ds(off, _CG)]], buf_v,
                             sem).wait()
            pltpu.sync_copy(buf_v, out_hbm.at[pl.ds(base + off, _CG)])
            return 0

        jax.lax.fori_loop(0, _BPW // _CG, chunk, 0)

    return _gather_sc


def kernel(ids, feats, edge_dict, epo, vcn_kk_w, vcn_kk_b, vcn_k1_w, vcn_k1_b,
           ec_w1, ec_b1, ec_w2, ec_b2, fc_w, fc_b):
    norms = jnp.sqrt(jnp.sum(feats * feats, axis=1, keepdims=True)) + 1e-12
    fn = feats / norms
    fn_pad = jnp.pad(fn, ((0, NP - N), (0, 0)))
    idxT_full = _simtopk(fn_pad)
    idxT = idxT_full[:, :N][:, ids]

    idxT_pad = jnp.pad(idxT, ((0, 0), (0, NVP - N)))
    g = _gather_sc_fn()(feats, idxT_pad.reshape(-1)).reshape(KN, NVP, D)

    wr = vcn_kk_w.reshape(KN, KN, D)
    out_full = _vconv(g, wr, vcn_kk_b.reshape(KN, KN), vcn_k1_w, vcn_k1_b,
                      fc_w.T, fc_b.reshape(1, D))
    return out_full[:N]


# speedup vs baseline: 9.1792x; 1.0566x over previous
import functools

import jax
import jax.numpy as jnp
from jax import lax
from jax.experimental import pallas as pl
from jax.experimental.pallas import tpu as pltpu
from jax.experimental.pallas import tpu_sc as plsc

N = 10000
D = 128
KN = 32
RB = 256
NRP = ((N + RB - 1) // RB) * RB
NP = 10240
G = 256
P = NP // G
RB2 = 256
NVP = ((N + RB2 - 1) // RB2) * RB2


_NEG = -3.0e38
_STALE = -2.0e38
_BIG = NP


def _insert3(state, v, cv):
    g1, c1, g2, c2, g3, c3 = state
    w1 = v > g1
    lv = jnp.where(w1, g1, v)
    lc = jnp.where(w1, c1, cv)
    g1 = jnp.where(w1, v, g1)
    c1 = jnp.where(w1, cv, c1)
    w2 = lv > g2
    lv2 = jnp.where(w2, g2, lv)
    lc2 = jnp.where(w2, c2, lc)
    g2 = jnp.where(w2, lv, g2)
    c2 = jnp.where(w2, lc, c2)
    w3 = lv2 > g3
    g3 = jnp.where(w3, lv2, g3)
    c3 = jnp.where(w3, lc2, c3)
    return (g1, c1, g2, c2, g3, c3)


def _init3():
    return (jnp.full((G, RB), _NEG, jnp.float32),
            jnp.full((G, RB), _BIG, jnp.int32),
            jnp.full((G, RB), _NEG, jnp.float32),
            jnp.full((G, RB), _BIG, jnp.int32),
            jnp.full((G, RB), _NEG, jnp.float32),
            jnp.full((G, RB), _BIG, jnp.int32))


def _simtopk_body(fn_ref, fnb_ref, idxT_ref, s_ref,
                  g1_ref, c1_ref, g2_ref, c2_ref, g3_ref, c3_ref):
    fnb = fnb_ref[...]
    gidx = jax.lax.broadcasted_iota(jnp.int32, (G, RB), 0)

    st = _init3()
    for p in range(P):
        v = jax.lax.dot_general(fn_ref[p * G:(p + 1) * G, :], fnb,
                                (((1,), (1,)), ((), ())),
                                preferred_element_type=jnp.float32)
        cv = gidx + (p * G)
        v = jnp.where(cv < N, v, _NEG)
        s_ref[p * G:(p + 1) * G, :] = v
        st = _insert3(st, v, cv)
    g1_ref[...], c1_ref[...], g2_ref[...], c2_ref[...], \
        g3_ref[...], c3_ref[...] = st

    krow = jax.lax.broadcasted_iota(jnp.int32, (KN, RB), 0)

    def ext(k, accT):
        g1 = g1_ref[...]
        c1 = c1_ref[...]
        m = jnp.max(g1, axis=0, keepdims=True)
        c = jnp.min(jnp.where(g1 == m, c1, _BIG), axis=0, keepdims=True)
        accT = jnp.where(krow == k, c, accT)
        hit = c1 == c
        g2 = g2_ref[...]
        need = jnp.any(hit & (g2 == _STALE))

        @pl.when(need)
        def _rebuild():
            st = _init3()
            for p in range(P):
                v = s_ref[p * G:(p + 1) * G, :]
                cv = gidx + (p * G)
                alive = (v < m) | ((v == m) & (cv > c))
                v = jnp.where(alive, v, _NEG)
                st = _insert3(st, v, cv)
            g1_ref[...], c1_ref[...], g2_ref[...], c2_ref[...], \
                g3_ref[...], c3_ref[...] = st

        @pl.when(jnp.logical_not(need))
        def _promote():
            g1_ref[...] = jnp.where(hit, g2, g1)
            c1_ref[...] = jnp.where(hit, c2_ref[...], c1)
            g2_ref[...] = jnp.where(hit, g3_ref[...], g2)
            c2_ref[...] = jnp.where(hit, c3_ref[...], c2_ref[...])
            g3_ref[...] = jnp.where(hit, _STALE, g3_ref[...])

        return accT

    idxT_ref[...] = jax.lax.fori_loop(
        0, KN, ext, jnp.zeros((KN, RB), jnp.int32))


def _simtopk(fn):
    return pl.pallas_call(
        _simtopk_body,
        grid=(NRP // RB,),
        in_specs=[
            pl.BlockSpec((NP, D), lambda i: (0, 0)),
            pl.BlockSpec((RB, D), lambda i: (i, 0)),
        ],
        out_specs=pl.BlockSpec((KN, RB), lambda i: (0, i)),
        out_shape=jax.ShapeDtypeStruct((KN, NRP), jnp.int32),
        scratch_shapes=[
            pltpu.VMEM((NP, RB), jnp.float32),
            pltpu.VMEM((G, RB), jnp.float32),
            pltpu.VMEM((G, RB), jnp.int32),
            pltpu.VMEM((G, RB), jnp.float32),
            pltpu.VMEM((G, RB), jnp.int32),
            pltpu.VMEM((G, RB), jnp.float32),
            pltpu.VMEM((G, RB), jnp.int32),
        ],
    )(fn, fn)


def _vconv_body(g_ref, wr_ref, kkb_ref, w1_ref, k1b_ref, fcwt_ref, fcb_ref,
                out_ref):
    kkbT = kkb_ref[...]
    cwT = jnp.zeros((KN, RB2), jnp.float32)
    for i in range(KN):
        convdT = jax.lax.dot_general(
            wr_ref[i], g_ref[i], (((1,), (1,)), ((), ())),
            preferred_element_type=jnp.float32)
        convdT = convdT + kkbT[:, i:i + 1]
        mx = jnp.max(convdT, axis=0, keepdims=True)
        e = jnp.exp(convdT - mx)
        ssum = jnp.sum(e, axis=0, keepdims=True)
        cwT = cwT + e * (w1_ref[i] / ssum)
    cw = cwT.T
    pooled = jnp.zeros((RB2, D), jnp.float32)
    for j in range(KN):
        pooled = pooled + cw[:, j:j + 1] * g_ref[j]
    pooled = pooled + k1b_ref[0]
    out = jnp.dot(pooled, fcwt_ref[...],
                  preferred_element_type=jnp.float32) + fcb_ref[...]
    out_ref[...] = jnp.maximum(out, 0.0)


def _vconv(g, wr, kkb, w1, k1b, fcwt, fcb):
    return pl.pallas_call(
        _vconv_body,
        grid=(NVP // RB2,),
        in_specs=[
            pl.BlockSpec((KN, RB2, D), lambda i: (0, i, 0)),
            pl.BlockSpec((KN, KN, D), lambda i: (0, 0, 0)),
            pl.BlockSpec((KN, KN), lambda i: (0, 0)),
            pl.BlockSpec(memory_space=pltpu.SMEM),
            pl.BlockSpec(memory_space=pltpu.SMEM),
            pl.BlockSpec((D, D), lambda i: (0, 0)),
            pl.BlockSpec((1, D), lambda i: (0, 0)),
        ],
        out_specs=pl.BlockSpec((RB2, D), lambda i: (i, 0)),
        out_shape=jax.ShapeDtypeStruct((NVP, D), jnp.float32),
    )(g, wr, kkb, w1, k1b, fcwt, fcb)


_NW = 32
_BG = KN * NVP
_BPW = _BG // _NW
_CG = 256


@functools.cache
def _gather_sc_fn():
    @functools.partial(
        pl.kernel,
        mesh=plsc.VectorSubcoreMesh(core_axis_name="c", subcore_axis_name="s"),
        out_type=jax.ShapeDtypeStruct((_BG, D), jnp.float32),
        scratch_types=[
            pltpu.VMEM((_BPW,), jnp.int32),
            pltpu.VMEM((_CG, D), jnp.float32),
            pltpu.SemaphoreType.DMA,
        ],
    )
    def _gather_sc(table_hbm, idx_hbm, out_hbm, idx_v, buf_v, sem):
        wid = lax.axis_index("s") * 2 + lax.axis_index("c")
        base = wid * _BPW
        pltpu.sync_copy(idx_hbm.at[pl.ds(base, _BPW)], idx_v)

        def chunk(ci, _):
            off = ci * _CG
            pltpu.async_copy(table_hbm.at[idx_v.at[pl.ds(off, _CG)]], buf_v,
                             sem).wait()
            pltpu.sync_copy(buf_v, out_hbm.at[pl.ds(base + off, _CG)])
            return 0

        jax.lax.fori_loop(0, _BPW // _CG, chunk, 0)

    return _gather_sc


def kernel(ids, feats, edge_dict, epo, vcn_kk_w, vcn_kk_b, vcn_k1_w, vcn_k1_b,
           ec_w1, ec_b1, ec_w2, ec_b2, fc_w, fc_b):
    norms = jnp.sqrt(jnp.sum(feats * feats, axis=1, keepdims=True)) + 1e-12
    fn = feats / norms
    fn_pad = jnp.pad(fn, ((0, NP - N), (0, 0)))
    idxT_full = _simtopk(fn_pad)
    idxT = idxT_full[:, :N][:, ids]

    idxT_pad = jnp.pad(idxT, ((0, 0), (0, NVP - N)))
    g = _gather_sc_fn()(feats, idxT_pad.reshape(-1)).reshape(KN, NVP, D)

    wr = vcn_kk_w.reshape(KN, KN, D)
    out_full = _vconv(g, wr, vcn_kk_b.reshape(KN, KN).T, vcn_k1_w, vcn_k1_b,
                      fc_w.T, fc_b.reshape(1, D))
    return out_full[:N]
